# baseline (speedup 1.0000x reference)
import jax
import jax.numpy as jnp
from jax import lax
from jax.experimental import pallas as pl
from jax.experimental.pallas import tpu as pltpu

NUM_BITS = 8
MIN_VAL = 0.0
MAX_VAL = 1.0
THRESHOLD = 3.0
B = 1024
F = 512
HIDDEN = 2048
CLASSES = 1000
ENC = F * NUM_BITS

BB = 128
HB0 = 128
HB1 = 256
HBC = 256

NB = B // BB
N0 = HIDDEN // HB0
N1 = HIDDEN // HB1
NC = 2 * HIDDEN // HBC
W0_DEPTH = 6
W1_DEPTH = 6
OC_DEPTH = 8
S_Z0 = NB
S_Z1 = S_Z0 + N0 // 2
S_OUT = S_Z1 + N1
S_ARGMAX = S_OUT + NC // 2
N_STEPS = S_ARGMAX + 1


def _w0_copy(w0_refs, w0r, w0_sem, blk, par, slot):
    return pltpu.make_async_copy(
        w0_refs[par].at[pl.ds(blk * HB0, HB0), :], w0r.at[slot],
        w0_sem.at[slot])


def _w1_copy(w1_ref, w1s, w1_sem, blk, slot):
    return pltpu.make_async_copy(
        w1_ref.at[pl.ds(blk * HB1, HB1), :], w1s.at[slot], w1_sem.at[slot])


def _oc_copy(oc_refs, ocr, oc_sem, par, layer, hblk, slot):
    return pltpu.make_async_copy(
        oc_refs[par].at[layer, pl.ds(hblk * HBC, HBC), :], ocr.at[slot],
        oc_sem.at[slot])


def _body(x_ref, w0a_ref, w0b_ref, w1_ref, oca_ref, ocb_ref,
          out_ref, pred_ref,
          enc_s, a0_s, a1_s, r_s, w0r, w1s, ocr, w0_sem, w1_sem, oc_sem):
    i = pl.program_id(0)
    w0_refs = (w0a_ref, w0b_ref)
    oc_refs = (oca_ref, ocb_ref)

    @pl.when(i == 0)
    def _():
        for b in range(W0_DEPTH):
            _w0_copy(w0_refs, w0r, w0_sem, b, b % 2, b).start()
        for b in range(W1_DEPTH):
            _w1_copy(w1_ref, w1s, w1_sem, b, b).start()
        for b in range(OC_DEPTH):
            _oc_copy(oc_refs, ocr, oc_sem, b % 2, 0, b, b).start()
        src = lax.broadcasted_iota(jnp.int32, (F, ENC), 1) // NUM_BITS
        dst = lax.broadcasted_iota(jnp.int32, (F, ENC), 0)
        r_s[...] = (src == dst).astype(jnp.bfloat16)

    @pl.when(i < S_Z0)
    def _():
        xb = x_ref[...]
        xc = jnp.clip(xb, MIN_VAL, MAX_VAL)
        norm = (xc - MIN_VAL) / (MAX_VAL - MIN_VAL)
        lv = jnp.round(norm * (2 ** NUM_BITS - 1)).astype(jnp.int32)
        gray = lv ^ (lv >> 1)
        rep = lax.dot_general(gray.astype(jnp.bfloat16), r_s[...],
                              (((1,), (0,)), ((), ())),
                              preferred_element_type=jnp.float32)
        gi = rep.astype(jnp.int32)
        kidx = lax.broadcasted_iota(jnp.int32, (BB, ENC), 1) & (NUM_BITS - 1)
        enc_s[pl.ds(i * BB, BB), :] = ((gi >> kidx) & 1).astype(jnp.bfloat16)

    @pl.when((i >= S_Z0) & (i < S_Z1))
    def _():
        s = i - S_Z0
        for par in range(2):
            b = s * 2 + par
            slot = lax.rem(b, W0_DEPTH)
            _w0_copy(w0_refs, w0r, w0_sem, b, par, slot).wait()
            wb = w0r[slot].astype(jnp.bfloat16)
            z = lax.dot_general(enc_s[...], wb, (((1,), (1,)), ((), ())),
                                preferred_element_type=jnp.float32)
            a0_s[:, pl.ds(b * HB0, HB0)] = (z >= THRESHOLD).astype(jnp.bfloat16)

            @pl.when(b + W0_DEPTH < N0)
            def _():
                _w0_copy(w0_refs, w0r, w0_sem, b + W0_DEPTH, par, slot).start()

    @pl.when((i >= S_Z1) & (i < S_OUT))
    def _():
        h = i - S_Z1
        slot = lax.rem(h, W1_DEPTH)
        _w1_copy(w1_ref, w1s, w1_sem, h, slot).wait()
        wb = w1s[slot].astype(jnp.bfloat16)
        z = lax.dot_general(a0_s[...], wb, (((1,), (1,)), ((), ())),
                            preferred_element_type=jnp.float32)
        a1_s[:, pl.ds(h * HB1, HB1)] = (z >= THRESHOLD).astype(jnp.bfloat16)

        @pl.when(h + W1_DEPTH < N1)
        def _():
            _w1_copy(w1_ref, w1s, w1_sem, h + W1_DEPTH, slot).start()

    @pl.when((i >= S_OUT) & (i < S_ARGMAX))
    def _():
        s = i - S_OUT
        for par in range(2):
            b = s * 2 + par
            layer = b // (NC // 2)
            h = lax.rem(b, NC // 2)
            slot = lax.rem(b, OC_DEPTH)
            _oc_copy(oc_refs, ocr, oc_sem, par, layer, h, slot).wait()
            ocb = ocr[slot]

            def acc(a_s):
                ab = a_s[:, pl.ds(h * HBC, HBC)].astype(jnp.float32)
                return lax.dot_general(ab, ocb, (((1,), (0,)), ((), ())),
                                       preferred_element_type=jnp.float32)

            @pl.when(b == 0)
            def _():
                out_ref[...] = acc(a0_s)

            @pl.when((b > 0) & (b < NC // 2))
            def _():
                out_ref[...] = out_ref[...] + acc(a0_s)

            @pl.when(b >= NC // 2)
            def _():
                out_ref[...] = out_ref[...] + acc(a1_s)

            @pl.when(b + OC_DEPTH < NC)
            def _():
                _oc_copy(oc_refs, ocr, oc_sem, par, 1, h, slot).start()

    @pl.when(i == S_ARGMAX)
    def _():
        out = out_ref[...]
        mx = jnp.max(out, axis=1, keepdims=True)
        idx = lax.broadcasted_iota(jnp.int32, out.shape, 1)
        pred = jnp.min(jnp.where(out == mx, idx, CLASSES), axis=1)
        pred_ref[...] = pred.reshape(NB, 1, BB).astype(jnp.int32)


def kernel(trainOrTest, x, y, W0, W1, outC):
    del trainOrTest, y

    out_act, preds3 = pl.pallas_call(
        _body,
        grid=(N_STEPS,),
        in_specs=[
            pl.BlockSpec((BB, F), lambda i: (jnp.minimum(i, NB - 1), 0)),
            pl.BlockSpec(memory_space=pl.ANY),
            pl.BlockSpec(memory_space=pl.ANY),
            pl.BlockSpec(memory_space=pl.ANY),
            pl.BlockSpec(memory_space=pl.ANY),
            pl.BlockSpec(memory_space=pl.ANY),
        ],
        out_specs=[
            pl.BlockSpec((B, CLASSES), lambda i: (0, 0)),
            pl.BlockSpec((NB, 1, BB), lambda i: (0, 0, 0)),
        ],
        out_shape=[
            jax.ShapeDtypeStruct((B, CLASSES), jnp.float32),
            jax.ShapeDtypeStruct((NB, 1, BB), jnp.int32),
        ],
        scratch_shapes=[
            pltpu.VMEM((B, ENC), jnp.bfloat16),
            pltpu.VMEM((B, HIDDEN), jnp.bfloat16),
            pltpu.VMEM((B, HIDDEN), jnp.bfloat16),
            pltpu.VMEM((F, ENC), jnp.bfloat16),
            pltpu.VMEM((W0_DEPTH, HB0, ENC), jnp.float32),
            pltpu.VMEM((W1_DEPTH, HB1, HIDDEN), jnp.float32),
            pltpu.VMEM((OC_DEPTH, HBC, CLASSES), jnp.float32),
            pltpu.SemaphoreType.DMA((W0_DEPTH,)),
            pltpu.SemaphoreType.DMA((W1_DEPTH,)),
            pltpu.SemaphoreType.DMA((OC_DEPTH,)),
        ],
    )(x, W0, W0, W1, outC, outC)

    predictions = preds3.reshape(B)
    return predictions, out_act


# speedup vs baseline: 1.0047x; 1.0047x over previous
import jax
import jax.numpy as jnp
from jax import lax
from jax.experimental import pallas as pl
from jax.experimental.pallas import tpu as pltpu

NUM_BITS = 8
MIN_VAL = 0.0
MAX_VAL = 1.0
THRESHOLD = 3.0
B = 1024
F = 512
HIDDEN = 2048
CLASSES = 1000
ENC = F * NUM_BITS

BB = 128
HB0 = 128
HB1 = 256
HBC = 256

NB = B // BB
N0 = HIDDEN // HB0
N1 = HIDDEN // HB1
NC = 2 * HIDDEN // HBC
W0_DEPTH = 6
W1_DEPTH = 6
OC_DEPTH = 8
S_Z0 = NB
S_Z1 = S_Z0 + N0 // 2
S_OUT = S_Z1 + N1
S_ARGMAX = S_OUT + NC // 2
N_STEPS = S_ARGMAX + 1


def _w0_copy(w0_refs, w0r, w0_sem, blk, par, slot):
    return pltpu.make_async_copy(
        w0_refs[par].at[pl.ds(blk * HB0, HB0), :], w0r.at[slot],
        w0_sem.at[slot])


def _w1_copy(w1_ref, w1s, w1_sem, blk, slot):
    return pltpu.make_async_copy(
        w1_ref.at[pl.ds(blk * HB1, HB1), :], w1s.at[slot], w1_sem.at[slot])


def _oc_copy(oc_refs, ocr, oc_sem, par, layer, hblk, slot):
    return pltpu.make_async_copy(
        oc_refs[par].at[layer, pl.ds(hblk * HBC, HBC), :], ocr.at[slot],
        oc_sem.at[slot])


def _body(x_ref, w0_ref, w1_ref, oc_ref,
          out_ref, pred_ref,
          enc_s, a0_s, a1_s, r_s, w0r, w1s, ocr, w0_sem, w1_sem, oc_sem):
    i = pl.program_id(0)
    w0_refs = (w0_ref, w0_ref)
    oc_refs = (oc_ref, oc_ref)

    @pl.when(i == 0)
    def _():
        for b in range(W0_DEPTH):
            _w0_copy(w0_refs, w0r, w0_sem, b, b % 2, b).start()
        for b in range(W1_DEPTH):
            _w1_copy(w1_ref, w1s, w1_sem, b, b).start()
        for b in range(OC_DEPTH):
            _oc_copy(oc_refs, ocr, oc_sem, b % 2, 0, b, b).start()
        src = lax.broadcasted_iota(jnp.int32, (F, ENC), 1) // NUM_BITS
        dst = lax.broadcasted_iota(jnp.int32, (F, ENC), 0)
        r_s[...] = (src == dst).astype(jnp.bfloat16)

    @pl.when(i < S_Z0)
    def _():
        xb = x_ref[...]
        xc = jnp.clip(xb, MIN_VAL, MAX_VAL)
        norm = (xc - MIN_VAL) / (MAX_VAL - MIN_VAL)
        lv = jnp.round(norm * (2 ** NUM_BITS - 1)).astype(jnp.int32)
        gray = lv ^ (lv >> 1)
        rep = lax.dot_general(gray.astype(jnp.bfloat16), r_s[...],
                              (((1,), (0,)), ((), ())),
                              preferred_element_type=jnp.float32)
        gi = rep.astype(jnp.int32)
        kidx = lax.broadcasted_iota(jnp.int32, (BB, ENC), 1) & (NUM_BITS - 1)
        enc_s[pl.ds(i * BB, BB), :] = ((gi >> kidx) & 1).astype(jnp.bfloat16)

    @pl.when((i >= S_Z0) & (i < S_Z1))
    def _():
        s = i - S_Z0
        for par in range(2):
            b = s * 2 + par
            slot = lax.rem(b, W0_DEPTH)
            _w0_copy(w0_refs, w0r, w0_sem, b, par, slot).wait()
            wb = w0r[slot].astype(jnp.bfloat16)
            z = lax.dot_general(enc_s[...], wb, (((1,), (1,)), ((), ())),
                                preferred_element_type=jnp.float32)
            a0_s[:, pl.ds(b * HB0, HB0)] = (z >= THRESHOLD).astype(jnp.bfloat16)

            @pl.when(b + W0_DEPTH < N0)
            def _():
                _w0_copy(w0_refs, w0r, w0_sem, b + W0_DEPTH, par, slot).start()

    @pl.when((i >= S_Z1) & (i < S_OUT))
    def _():
        h = i - S_Z1
        slot = lax.rem(h, W1_DEPTH)
        _w1_copy(w1_ref, w1s, w1_sem, h, slot).wait()
        wb = w1s[slot].astype(jnp.bfloat16)
        z = lax.dot_general(a0_s[...], wb, (((1,), (1,)), ((), ())),
                            preferred_element_type=jnp.float32)
        a1_s[:, pl.ds(h * HB1, HB1)] = (z >= THRESHOLD).astype(jnp.bfloat16)

        @pl.when(h + W1_DEPTH < N1)
        def _():
            _w1_copy(w1_ref, w1s, w1_sem, h + W1_DEPTH, slot).start()

    @pl.when((i >= S_OUT) & (i < S_ARGMAX))
    def _():
        s = i - S_OUT
        for par in range(2):
            b = s * 2 + par
            layer = b // (NC // 2)
            h = lax.rem(b, NC // 2)
            slot = lax.rem(b, OC_DEPTH)
            _oc_copy(oc_refs, ocr, oc_sem, par, layer, h, slot).wait()
            ocb = ocr[slot]

            def acc(a_s):
                ab = a_s[:, pl.ds(h * HBC, HBC)].astype(jnp.float32)
                return lax.dot_general(ab, ocb, (((1,), (0,)), ((), ())),
                                       preferred_element_type=jnp.float32)

            @pl.when(b == 0)
            def _():
                out_ref[...] = acc(a0_s)

            @pl.when((b > 0) & (b < NC // 2))
            def _():
                out_ref[...] = out_ref[...] + acc(a0_s)

            @pl.when(b >= NC // 2)
            def _():
                out_ref[...] = out_ref[...] + acc(a1_s)

            @pl.when(b + OC_DEPTH < NC)
            def _():
                _oc_copy(oc_refs, ocr, oc_sem, par, 1, h, slot).start()

    @pl.when(i == S_ARGMAX)
    def _():
        out = out_ref[...]
        mx = jnp.max(out, axis=1, keepdims=True)
        idx = lax.broadcasted_iota(jnp.int32, out.shape, 1)
        pred = jnp.min(jnp.where(out == mx, idx, CLASSES), axis=1)
        pred_ref[...] = pred.reshape(NB, 1, BB).astype(jnp.int32)


def kernel(trainOrTest, x, y, W0, W1, outC):
    del trainOrTest, y

    out_act, preds3 = pl.pallas_call(
        _body,
        grid=(N_STEPS,),
        in_specs=[
            pl.BlockSpec((BB, F), lambda i: (jnp.minimum(i, NB - 1), 0)),
            pl.BlockSpec(memory_space=pl.ANY),
            pl.BlockSpec(memory_space=pl.ANY),
            pl.BlockSpec(memory_space=pl.ANY),
        ],
        out_specs=[
            pl.BlockSpec((B, CLASSES), lambda i: (0, 0)),
            pl.BlockSpec((NB, 1, BB), lambda i: (0, 0, 0)),
        ],
        out_shape=[
            jax.ShapeDtypeStruct((B, CLASSES), jnp.float32),
            jax.ShapeDtypeStruct((NB, 1, BB), jnp.int32),
        ],
        scratch_shapes=[
            pltpu.VMEM((B, ENC), jnp.bfloat16),
            pltpu.VMEM((B, HIDDEN), jnp.bfloat16),
            pltpu.VMEM((B, HIDDEN), jnp.bfloat16),
            pltpu.VMEM((F, ENC), jnp.bfloat16),
            pltpu.VMEM((W0_DEPTH, HB0, ENC), jnp.float32),
            pltpu.VMEM((W1_DEPTH, HB1, HIDDEN), jnp.float32),
            pltpu.VMEM((OC_DEPTH, HBC, CLASSES), jnp.float32),
            pltpu.SemaphoreType.DMA((W0_DEPTH,)),
            pltpu.SemaphoreType.DMA((W1_DEPTH,)),
            pltpu.SemaphoreType.DMA((OC_DEPTH,)),
        ],
    )(x, W0, W1, outC)

    predictions = preds3.reshape(B)
    return predictions, out_act


# speedup vs baseline: 1.1595x; 1.1541x over previous
import jax
import jax.numpy as jnp
from jax import lax
from jax.experimental import pallas as pl
from jax.experimental.pallas import tpu as pltpu

NUM_BITS = 8
MIN_VAL = 0.0
MAX_VAL = 1.0
THRESHOLD = 3.0
B = 1024
F = 512
HIDDEN = 2048
CLASSES = 1000
ENC = F * NUM_BITS

BB = 128
HB = 128

NB = B // BB
NH = HIDDEN // HB
S_Z0 = NB
S_Z1 = S_Z0 + NH // 2
S_OUT = S_Z1 + NH // 2
S_ARGMAX = S_OUT + NH
N_STEPS = S_ARGMAX + 1


def _body(x_ref, w0a_ref, w0b_ref, w1a_ref, w1b_ref, oca_ref, ocb_ref,
          out_ref, pred_ref, enc_s, a0_s, a1_s, r_s):
    i = pl.program_id(0)

    @pl.when(i == 0)
    def _():
        src = lax.broadcasted_iota(jnp.int32, (F, ENC), 1) // NUM_BITS
        dst = lax.broadcasted_iota(jnp.int32, (F, ENC), 0)
        r_s[...] = (src == dst).astype(jnp.bfloat16)

    @pl.when(i < S_Z0)
    def _():
        xb = x_ref[...]
        xc = jnp.clip(xb, MIN_VAL, MAX_VAL)
        norm = (xc - MIN_VAL) / (MAX_VAL - MIN_VAL)
        lv = jnp.round(norm * (2 ** NUM_BITS - 1)).astype(jnp.int32)
        gray = lv ^ (lv >> 1)
        rep = lax.dot_general(gray.astype(jnp.bfloat16), r_s[...],
                              (((1,), (0,)), ((), ())),
                              preferred_element_type=jnp.float32)
        gi = rep.astype(jnp.int32)
        kidx = lax.broadcasted_iota(jnp.int32, (BB, ENC), 1) & (NUM_BITS - 1)
        enc_s[pl.ds(i * BB, BB), :] = ((gi >> kidx) & 1).astype(jnp.bfloat16)

    def layer_step(step0, act_s, wa_ref, wb_ref, dst_s):
        h2 = (i - step0) * 2
        for h, wref in ((h2, wa_ref), (h2 + 1, wb_ref)):
            wb = wref[...].astype(jnp.bfloat16)
            z = lax.dot_general(act_s[...], wb, (((1,), (1,)), ((), ())),
                                preferred_element_type=jnp.float32)
            dst_s[:, pl.ds(h * HB, HB)] = (z >= THRESHOLD).astype(jnp.bfloat16)

    @pl.when((i >= S_Z0) & (i < S_Z1))
    def _():
        layer_step(S_Z0, enc_s, w0a_ref, w0b_ref, a0_s)

    @pl.when((i >= S_Z1) & (i < S_OUT))
    def _():
        layer_step(S_Z1, a0_s, w1a_ref, w1b_ref, a1_s)

    @pl.when((i >= S_OUT) & (i < S_ARGMAX))
    def _():
        j = i - S_OUT
        h2 = jnp.where(j < NH // 2, j, j - NH // 2) * 2

        def acc(a_s):
            p = lax.dot_general(
                a_s[:, pl.ds(h2 * HB, HB)].astype(jnp.float32), oca_ref[0],
                (((1,), (0,)), ((), ())), preferred_element_type=jnp.float32)
            return p + lax.dot_general(
                a_s[:, pl.ds((h2 + 1) * HB, HB)].astype(jnp.float32),
                ocb_ref[0],
                (((1,), (0,)), ((), ())), preferred_element_type=jnp.float32)

        @pl.when(j < NH // 2)
        def _():
            p = acc(a0_s)

            @pl.when(j == 0)
            def _():
                out_ref[...] = p

            @pl.when(j > 0)
            def _():
                out_ref[...] = out_ref[...] + p

        @pl.when(j >= NH // 2)
        def _():
            out_ref[...] = out_ref[...] + acc(a1_s)

    @pl.when(i == S_ARGMAX)
    def _():
        out = out_ref[...]
        mx = jnp.max(out, axis=1, keepdims=True)
        idx = lax.broadcasted_iota(jnp.int32, out.shape, 1)
        pred = jnp.min(jnp.where(out == mx, idx, CLASSES), axis=1)
        pred_ref[...] = pred.reshape(NB, 1, BB).astype(jnp.int32)


def kernel(trainOrTest, x, y, W0, W1, outC):
    del trainOrTest, y

    def w_index(step0, off):
        def f(i):
            return (jnp.clip(i - step0, 0, NH // 2 - 1) * 2 + off, 0)
        return f

    def oc_index(off):
        def f(i):
            j = jnp.clip(i - S_OUT, 0, NH - 1)
            layer = j // (NH // 2)
            h2 = jnp.where(j < NH // 2, j, j - NH // 2) * 2
            return (layer, h2 + off, 0)
        return f

    out_act, preds3 = pl.pallas_call(
        _body,
        grid=(N_STEPS,),
        in_specs=[
            pl.BlockSpec((BB, F), lambda i: (jnp.minimum(i, NB - 1), 0)),
            pl.BlockSpec((HB, ENC), w_index(S_Z0, 0)),
            pl.BlockSpec((HB, ENC), w_index(S_Z0, 1)),
            pl.BlockSpec((HB, HIDDEN), w_index(S_Z1, 0)),
            pl.BlockSpec((HB, HIDDEN), w_index(S_Z1, 1)),
            pl.BlockSpec((1, HB, CLASSES), oc_index(0)),
            pl.BlockSpec((1, HB, CLASSES), oc_index(1)),
        ],
        out_specs=[
            pl.BlockSpec((B, CLASSES), lambda i: (0, 0)),
            pl.BlockSpec((NB, 1, BB), lambda i: (0, 0, 0)),
        ],
        out_shape=[
            jax.ShapeDtypeStruct((B, CLASSES), jnp.float32),
            jax.ShapeDtypeStruct((NB, 1, BB), jnp.int32),
        ],
        scratch_shapes=[
            pltpu.VMEM((B, ENC), jnp.bfloat16),
            pltpu.VMEM((B, HIDDEN), jnp.bfloat16),
            pltpu.VMEM((B, HIDDEN), jnp.bfloat16),
            pltpu.VMEM((F, ENC), jnp.bfloat16),
        ],
    )(x, W0, W0, W1, W1, outC, outC)

    predictions = preds3.reshape(B)
    return predictions, out_act


# speedup vs baseline: 1.7554x; 1.5139x over previous
import jax
import jax.numpy as jnp
from jax import lax
from jax.experimental import pallas as pl
from jax.experimental.pallas import tpu as pltpu

NUM_BITS = 8
MIN_VAL = 0.0
MAX_VAL = 1.0
THRESHOLD = 3.0
B = 1024
F = 512
HIDDEN = 2048
CLASSES = 1000
ENC = F * NUM_BITS

BB = 256
HB = 256
HC = 256

NB = B // BB
NH = HIDDEN // HB
NCL = HIDDEN // HC
S_Z0 = NB
S_Z1 = S_Z0 + NH // 2
S_OUT = S_Z1 + NH // 2
N_OUT = NCL
N_STEPS = S_OUT + N_OUT


def _body(x_ref, w0a_ref, w0b_ref, w1a_ref, w1b_ref, oca_ref, ocb_ref,
          out_ref, pred_ref, enc_s, a0_s, a1_s, r_s):
    i = pl.program_id(0)

    @pl.when(i == 0)
    def _():
        src = lax.broadcasted_iota(jnp.int32, (F, ENC), 1) // NUM_BITS
        dst = lax.broadcasted_iota(jnp.int32, (F, ENC), 0)
        r_s[...] = (src == dst).astype(jnp.bfloat16)

    @pl.when(i < S_Z0)
    def _():
        xb = x_ref[...]
        xc = jnp.clip(xb, MIN_VAL, MAX_VAL)
        norm = (xc - MIN_VAL) / (MAX_VAL - MIN_VAL)
        lv = jnp.round(norm * (2 ** NUM_BITS - 1)).astype(jnp.int32)
        gray = lv ^ (lv >> 1)
        rep = lax.dot_general(gray.astype(jnp.bfloat16), r_s[...],
                              (((1,), (0,)), ((), ())),
                              preferred_element_type=jnp.float32)
        gi = rep.astype(jnp.int32)
        kidx = lax.broadcasted_iota(jnp.int32, (BB, ENC), 1) & (NUM_BITS - 1)
        enc_s[pl.ds(i * BB, BB), :] = ((gi >> kidx) & 1).astype(jnp.bfloat16)

    def layer_step(step0, act_s, wa_ref, wb_ref, dst_s):
        h2 = (i - step0) * 2
        for h, wref in ((h2, wa_ref), (h2 + 1, wb_ref)):
            wb = wref[...].astype(jnp.bfloat16)
            z = lax.dot_general(act_s[...], wb, (((1,), (1,)), ((), ())),
                                preferred_element_type=jnp.float32)
            dst_s[:, pl.ds(h * HB, HB)] = (z >= THRESHOLD).astype(jnp.bfloat16)

    @pl.when((i >= S_Z0) & (i < S_Z1))
    def _():
        layer_step(S_Z0, enc_s, w0a_ref, w0b_ref, a0_s)

    @pl.when((i >= S_Z1) & (i < S_OUT))
    def _():
        layer_step(S_Z1, a0_s, w1a_ref, w1b_ref, a1_s)

    @pl.when(i >= S_OUT)
    def _():
        j = i - S_OUT
        hba = 2 * lax.rem(j, N_OUT // 2)

        def acc(a_s):
            p = lax.dot_general(
                a_s[:, pl.ds(hba * HC, HC)].astype(jnp.float32), oca_ref[0],
                (((1,), (0,)), ((), ())), preferred_element_type=jnp.float32)
            return p + lax.dot_general(
                a_s[:, pl.ds((hba + 1) * HC, HC)].astype(jnp.float32),
                ocb_ref[0],
                (((1,), (0,)), ((), ())), preferred_element_type=jnp.float32)

        @pl.when(j == 0)
        def _():
            out_ref[...] = acc(a0_s)

        @pl.when((j > 0) & (j < N_OUT // 2))
        def _():
            out_ref[...] = out_ref[...] + acc(a0_s)

        @pl.when(j >= N_OUT // 2)
        def _():
            out_ref[...] = out_ref[...] + acc(a1_s)

        @pl.when(j == N_OUT - 1)
        def _():
            out = out_ref[...]
            mx = jnp.max(out, axis=1, keepdims=True)
            idx = lax.broadcasted_iota(jnp.int32, out.shape, 1)
            pred = jnp.min(jnp.where(out == mx, idx, CLASSES), axis=1)
            pred_ref[...] = pred.reshape(NB, 1, BB).astype(jnp.int32)


def kernel(trainOrTest, x, y, W0, W1, outC):
    del trainOrTest, y

    def w_index(step0, off):
        def f(i):
            return (jnp.clip(i - step0, 0, NH // 2 - 1) * 2 + off, 0)
        return f

    def oc_index(off):
        def f(i):
            j = jnp.clip(i - S_OUT, 0, N_OUT - 1)
            return (j // (N_OUT // 2), 2 * (j % (N_OUT // 2)) + off, 0)
        return f

    out_act, preds3 = pl.pallas_call(
        _body,
        grid=(N_STEPS,),
        in_specs=[
            pl.BlockSpec((BB, F), lambda i: (jnp.minimum(i, NB - 1), 0)),
            pl.BlockSpec((HB, ENC), w_index(S_Z0, 0)),
            pl.BlockSpec((HB, ENC), w_index(S_Z0, 1)),
            pl.BlockSpec((HB, HIDDEN), w_index(S_Z1, 0)),
            pl.BlockSpec((HB, HIDDEN), w_index(S_Z1, 1)),
            pl.BlockSpec((1, HC, CLASSES), oc_index(0)),
            pl.BlockSpec((1, HC, CLASSES), oc_index(1)),
        ],
        out_specs=[
            pl.BlockSpec((B, CLASSES), lambda i: (0, 0)),
            pl.BlockSpec((NB, 1, BB), lambda i: (0, 0, 0)),
        ],
        out_shape=[
            jax.ShapeDtypeStruct((B, CLASSES), jnp.float32),
            jax.ShapeDtypeStruct((NB, 1, BB), jnp.int32),
        ],
        scratch_shapes=[
            pltpu.VMEM((B, ENC), jnp.bfloat16),
            pltpu.VMEM((B, HIDDEN), jnp.bfloat16),
            pltpu.VMEM((B, HIDDEN), jnp.bfloat16),
            pltpu.VMEM((F, ENC), jnp.bfloat16),
        ],
    )(x, W0, W0, W1, W1, outC, outC)

    predictions = preds3.reshape(B)
    return predictions, out_act
